# baseline (speedup 1.0000x reference)
import functools

import jax
import jax.numpy as jnp
from jax.experimental import pallas as pl

_T = 128
_PP = 24


def _me_kernel(px, py, vx, vy, ori, msk, cx, cy,
               ptype, proute, ptl, hassl, slval,
               W1, b1, W2, b2, W3a, W3b, b3, W4, b4,
               slW1, slb1, slW2, slb2, temb, remb, tlemb, uemb,
               out):
    T, PP = px.shape
    f32 = jnp.float32

    relx = px[...] - cx[...]
    rely = py[...] - cy[...]
    co = jnp.cos(ori[...])
    si = jnp.sin(ori[...])
    m = msk[...]

    W1v = W1[...]

    def r1(row):
        return row.reshape(1, 1, -1)

    h1 = (relx[..., None] * r1(W1v[0])
          + rely[..., None] * r1(W1v[1])
          + vx[...][..., None] * r1(W1v[2])
          + vy[...][..., None] * r1(W1v[3])
          + co[..., None] * r1(W1v[4])
          + si[..., None] * r1(W1v[5])
          + b1[...].reshape(1, 1, -1))
    h1 = jnp.maximum(h1, 0.0)

    hf = jnp.dot(h1.reshape(T * PP, 128), W2[...],
                 preferred_element_type=f32) + b2[...]
    h3 = hf.reshape(T, PP, 256) * m[..., None]
    pooled = jnp.max(h3, axis=1)

    u = jnp.dot(h3.reshape(T * PP, 256), W3a[...],
                preferred_element_type=f32)
    pw = jnp.dot(pooled, W3b[...], preferred_element_type=f32)
    u3 = u.reshape(T, PP, 256) + pw[:, None, :] + b3[...].reshape(1, 1, -1)
    u3 = jnp.maximum(u3, 0.0)

    g = jnp.dot(u3.reshape(T * PP, 256), W4[...],
                preferred_element_type=f32) + b4[...]
    g3 = g.reshape(T, PP, 128) * m[..., None]
    acc = jnp.max(g3, axis=1)

    tt = ptype[...]
    for k in range(3):
        acc = acc + jnp.where(tt == k, 1.0, 0.0) * temb[k][None, :]
    rr = proute[...]
    for k in range(2):
        acc = acc + jnp.where(rr == k, 1.0, 0.0) * remb[k][None, :]
    tl = ptl[...]
    for k in range(4):
        acc = acc + jnp.where(tl == k, 1.0, 0.0) * tlemb[k][None, :]

    s = slval[...]
    s1 = jnp.maximum(s * slW1[...] + slb1[...], 0.0)
    sl2 = jnp.dot(s1, slW2[...], preferred_element_type=f32) + slb2[...]
    hs = hassl[...]
    acc = acc + hs * sl2 + (1.0 - hs) * uemb[...]

    out[...] = acc


def kernel(polygon_center, polygon_type, polygon_on_route, polygon_tl_status,
           polygon_has_speed_limit, polygon_speed_limit, point_position,
           point_vector, point_orientation, valid_mask,
           pe_W1, pe_b1, pe_W2, pe_b2, pe_W3, pe_b3, pe_W4, pe_b4,
           sl_W1, sl_b1, sl_W2, sl_b2, type_emb, on_route_emb, tl_emb,
           unknown_speed_emb):
    bs, M = polygon_type.shape
    P = valid_mask.shape[-1]
    N = bs * M
    dim = pe_W4.shape[-1]

    def padP(a):
        return jnp.pad(a, ((0, 0), (0, _PP - P)))

    px = padP(point_position[:, :, 0, :, 0].reshape(N, P))
    py = padP(point_position[:, :, 0, :, 1].reshape(N, P))
    vx = padP(point_vector[:, :, 0, :, 0].reshape(N, P))
    vy = padP(point_vector[:, :, 0, :, 1].reshape(N, P))
    ori = padP(point_orientation[:, :, 0].reshape(N, P))
    msk = padP(valid_mask.reshape(N, P).astype(jnp.float32))
    cx = polygon_center[..., 0].reshape(N, 1)
    cy = polygon_center[..., 1].reshape(N, 1)
    ptype = polygon_type.reshape(N, 1).astype(jnp.int32)
    proute = polygon_on_route.reshape(N, 1).astype(jnp.int32)
    ptl = polygon_tl_status.reshape(N, 1).astype(jnp.int32)
    hassl = polygon_has_speed_limit.reshape(N, 1).astype(jnp.float32)
    slval = polygon_speed_limit.reshape(N, 1)
    W3a = pe_W3[:256]
    W3b = pe_W3[256:]
    b1 = pe_b1.reshape(1, -1)
    b2 = pe_b2.reshape(1, -1)
    b3 = pe_b3.reshape(1, -1)
    b4 = pe_b4.reshape(1, -1)
    slb1 = sl_b1.reshape(1, -1)
    slb2 = sl_b2.reshape(1, -1)

    grid = (N // _T,)

    def tok(shape_last):
        return pl.BlockSpec((_T, shape_last), lambda i: (i, 0))

    def full(a):
        return pl.BlockSpec(a.shape, lambda i: (0,) * a.ndim)

    in_specs = [
        tok(_PP), tok(_PP), tok(_PP), tok(_PP), tok(_PP), tok(_PP),
        tok(1), tok(1), tok(1), tok(1), tok(1), tok(1), tok(1),
        full(pe_W1), full(b1), full(pe_W2), full(b2),
        full(W3a), full(W3b), full(b3), full(pe_W4), full(b4),
        full(sl_W1), full(slb1), full(sl_W2), full(slb2),
        full(type_emb), full(on_route_emb), full(tl_emb),
        full(unknown_speed_emb),
    ]

    out = pl.pallas_call(
        _me_kernel,
        grid=grid,
        in_specs=in_specs,
        out_specs=pl.BlockSpec((_T, dim), lambda i: (i, 0)),
        out_shape=jax.ShapeDtypeStruct((N, dim), jnp.float32),
    )(px, py, vx, vy, ori, msk, cx, cy, ptype, proute, ptl, hassl, slval,
      pe_W1, b1, pe_W2, b2, W3a, W3b, b3, pe_W4, b4,
      sl_W1, slb1, sl_W2, slb2, type_emb, on_route_emb, tl_emb,
      unknown_speed_emb)

    return out.reshape(bs, M, dim)


# speedup vs baseline: 2.6086x; 2.6086x over previous
import jax
import jax.numpy as jnp
from jax.experimental import pallas as pl

_T = 512
_P = 20


def _me_kernel(relx, rely, vx, vy, ori, msk,
               cidx, hassl, slval,
               W1a, W2a, W3aC, W3ba, W4a,
               slW1, slb1, slW2, slb2, ctab, uemb,
               out):
    f32 = jnp.float32
    bf16 = jnp.bfloat16
    PT = _P * _T

    def row(ref):
        return ref[...].reshape(1, PT)

    ones = jnp.ones((1, PT), dtype=bf16)
    feat = jnp.concatenate(
        [row(relx), row(rely), row(vx), row(vy),
         jnp.cos(row(ori)).astype(bf16), jnp.sin(row(ori)).astype(bf16)],
        axis=0)
    feat = jnp.concatenate([feat, ones], axis=0)
    m = row(msk)

    h1m = (jnp.maximum(jnp.dot(W1a[...], feat, preferred_element_type=f32),
                       0.0) * m.astype(f32)).astype(bf16)
    hm = jnp.dot(W2a[...], h1m, preferred_element_type=f32)

    pooled = hm[:, 0:_T]
    for p in range(1, _P):
        pooled = jnp.maximum(pooled, hm[:, p * _T:(p + 1) * _T])

    pw = jnp.dot(W3ba[...], pooled.astype(bf16),
                 preferred_element_type=f32)
    ud = jnp.dot(W3aC[...], h1m, preferred_element_type=f32)
    u = jnp.concatenate(
        [jnp.maximum(ud[:, p * _T:(p + 1) * _T] + pw, 0.0).astype(bf16)
         for p in range(_P)], axis=1)
    g = (jnp.dot(W4a[...], u, preferred_element_type=f32)
         * m.astype(f32))

    gp = g[:, 0:_T]
    for p in range(1, _P):
        gp = jnp.maximum(gp, g[:, p * _T:(p + 1) * _T])

    acc = gp.T

    ci = cidx[...]
    iota = jax.lax.broadcasted_iota(jnp.int32, (_T, 24), 1)
    oh = jnp.where(iota == ci, 1.0, 0.0)
    acc = acc + jnp.dot(oh, ctab[...], preferred_element_type=f32)

    s = slval[...]
    s1 = jnp.maximum(s * slW1[...] + slb1[...], 0.0)
    sl2 = jnp.dot(s1, slW2[...], preferred_element_type=f32) + slb2[...]
    hs = hassl[...]
    acc = acc + hs * sl2 + (1.0 - hs) * uemb[...]

    out[...] = acc


def kernel(polygon_center, polygon_type, polygon_on_route, polygon_tl_status,
           polygon_has_speed_limit, polygon_speed_limit, point_position,
           point_vector, point_orientation, valid_mask,
           pe_W1, pe_b1, pe_W2, pe_b2, pe_W3, pe_b3, pe_W4, pe_b4,
           sl_W1, sl_b1, sl_W2, sl_b2, type_emb, on_route_emb, tl_emb,
           unknown_speed_emb):
    bs, M = polygon_type.shape
    P = valid_mask.shape[-1]
    N = bs * M
    Nb = N // _T
    dim = pe_W4.shape[-1]
    f32 = jnp.float32
    bf16 = jnp.bfloat16

    def pm(a):
        return a.reshape(Nb, _T, P).transpose(0, 2, 1).reshape(Nb, 1, P * _T)

    relx = pm(point_position[:, :, 0, :, 0].reshape(N, P)
              - polygon_center[..., 0].reshape(N, 1)).astype(bf16)
    rely = pm(point_position[:, :, 0, :, 1].reshape(N, P)
              - polygon_center[..., 1].reshape(N, 1)).astype(bf16)
    vx = pm(point_vector[:, :, 0, :, 0].reshape(N, P)).astype(bf16)
    vy = pm(point_vector[:, :, 0, :, 1].reshape(N, P)).astype(bf16)
    ori = pm(point_orientation[:, :, 0].reshape(N, P))
    msk = pm(valid_mask.reshape(N, P).astype(f32)).astype(bf16)

    cidx = (polygon_type * 8 + polygon_tl_status * 2
            + polygon_on_route).reshape(N, 1).astype(jnp.int32)
    hassl = polygon_has_speed_limit.reshape(N, 1).astype(f32)
    slval = polygon_speed_limit.reshape(N, 1)

    d1 = pe_W1.shape[1]
    d2 = pe_W2.shape[1]
    din = pe_W1.shape[0]
    e1 = jnp.zeros((1, din + 1), f32).at[0, din].set(1.0)
    W1a = jnp.concatenate(
        [jnp.concatenate([pe_W1.T, pe_b1[:, None]], axis=1), e1],
        axis=0).astype(bf16)
    e2 = jnp.zeros((1, d2 + 1), f32).at[0, d2].set(1.0)
    W2af = jnp.concatenate(
        [jnp.concatenate([pe_W2.T, pe_b2[:, None]], axis=1),
         jnp.zeros((1, d1 + 1), f32).at[0, d1].set(1.0)],
        axis=0)
    W2a = W2af.astype(bf16)
    W3aa = jnp.concatenate(
        [jnp.concatenate([pe_W3[:d2].T, jnp.zeros((d2, 1), f32)], axis=1),
         jnp.zeros((1, d2 + 1), f32)], axis=0)
    W3aC = (W3aa @ W2af).astype(bf16)
    W3ba = jnp.concatenate(
        [jnp.concatenate([pe_W3[d2:].T, pe_b3[:, None]], axis=1),
         e2], axis=0).astype(bf16)
    W4a = jnp.concatenate([pe_W4.T, pe_b4[:, None]],
                          axis=1).astype(bf16)

    ctab = (type_emb[:, None, None, :] + tl_emb[None, :, None, :]
            + on_route_emb[None, None, :, :]).reshape(24, dim)

    slb1 = sl_b1.reshape(1, -1)
    slb2 = sl_b2.reshape(1, -1)
    slW2b = sl_W2

    grid = (Nb,)
    PT = P * _T

    def pmspec():
        return pl.BlockSpec((1, 1, PT), lambda i: (i, 0, 0))

    def tok():
        return pl.BlockSpec((_T, 1), lambda i: (i, 0))

    def full(a):
        return pl.BlockSpec(a.shape, lambda i: (0,) * a.ndim)

    in_specs = [
        pmspec(), pmspec(), pmspec(), pmspec(), pmspec(), pmspec(),
        tok(), tok(), tok(),
        full(W1a), full(W2a), full(W3aC), full(W3ba), full(W4a),
        full(sl_W1), full(slb1), full(slW2b), full(slb2),
        full(ctab), full(unknown_speed_emb),
    ]

    out = pl.pallas_call(
        _me_kernel,
        grid=grid,
        in_specs=in_specs,
        out_specs=pl.BlockSpec((_T, dim), lambda i: (i, 0)),
        out_shape=jax.ShapeDtypeStruct((N, dim), jnp.float32),
    )(relx, rely, vx, vy, ori, msk, cidx, hassl, slval,
      W1a, W2a, W3aC, W3ba, W4a,
      sl_W1, slb1, slW2b, slb2, ctab, unknown_speed_emb)

    return out.reshape(bs, M, dim)


# speedup vs baseline: 2.6203x; 1.0045x over previous
import jax
import jax.numpy as jnp
from jax.experimental import pallas as pl

_T = 512
_P = 20


def _me_kernel(comps, ori,
               cidx, hassl, slval,
               W1a, W2a, W3aC, W3ba, W4a,
               slW1, slb1, slW2, slb2, ctab, uemb,
               out):
    f32 = jnp.float32
    bf16 = jnp.bfloat16
    PT = _P * _T

    def row(ref):
        return ref[...].reshape(1, PT)

    ones = jnp.ones((1, PT), dtype=bf16)
    cb = comps[...].reshape(5, PT)
    feat = jnp.concatenate(
        [cb[0:4],
         jnp.cos(row(ori)).astype(bf16), jnp.sin(row(ori)).astype(bf16),
         ones], axis=0)
    m = cb[4:5]

    h1m = (jnp.maximum(jnp.dot(W1a[...], feat, preferred_element_type=f32),
                       0.0) * m.astype(f32)).astype(bf16)
    hm = jnp.dot(W2a[...], h1m, preferred_element_type=f32)

    pooled = hm[:, 0:_T]
    for p in range(1, _P):
        pooled = jnp.maximum(pooled, hm[:, p * _T:(p + 1) * _T])

    pw = jnp.dot(W3ba[...], pooled.astype(bf16),
                 preferred_element_type=f32)
    ud = jnp.dot(W3aC[...], h1m, preferred_element_type=f32)
    u = jnp.concatenate(
        [jnp.maximum(ud[:, p * _T:(p + 1) * _T] + pw, 0.0).astype(bf16)
         for p in range(_P)], axis=1)
    g = (jnp.dot(W4a[...], u, preferred_element_type=f32)
         * m.astype(f32))

    gp = g[:, 0:_T]
    for p in range(1, _P):
        gp = jnp.maximum(gp, g[:, p * _T:(p + 1) * _T])

    acc = gp.T

    ci = cidx[...]
    iota = jax.lax.broadcasted_iota(jnp.int32, (_T, 24), 1)
    oh = jnp.where(iota == ci, 1.0, 0.0)
    acc = acc + jnp.dot(oh, ctab[...], preferred_element_type=f32)

    s = slval[...]
    s1 = jnp.maximum(s * slW1[...] + slb1[...], 0.0)
    sl2 = jnp.dot(s1, slW2[...], preferred_element_type=f32) + slb2[...]
    hs = hassl[...]
    acc = acc + hs * sl2 + (1.0 - hs) * uemb[...]

    out[...] = acc


def kernel(polygon_center, polygon_type, polygon_on_route, polygon_tl_status,
           polygon_has_speed_limit, polygon_speed_limit, point_position,
           point_vector, point_orientation, valid_mask,
           pe_W1, pe_b1, pe_W2, pe_b2, pe_W3, pe_b3, pe_W4, pe_b4,
           sl_W1, sl_b1, sl_W2, sl_b2, type_emb, on_route_emb, tl_emb,
           unknown_speed_emb):
    bs, M = polygon_type.shape
    P = valid_mask.shape[-1]
    N = bs * M
    Nb = N // _T
    dim = pe_W4.shape[-1]
    f32 = jnp.float32
    bf16 = jnp.bfloat16

    rel = (point_position[:, :, 0].reshape(N, P, 2)
           - polygon_center[..., None, :2].reshape(N, 1, 2))
    comps = jnp.stack(
        [rel[..., 0], rel[..., 1],
         point_vector[:, :, 0, :, 0].reshape(N, P),
         point_vector[:, :, 0, :, 1].reshape(N, P),
         valid_mask.reshape(N, P).astype(f32)], axis=0).astype(bf16)
    comps = (comps.reshape(5, Nb, _T, P).transpose(1, 0, 3, 2)
             .reshape(Nb, 5, P * _T))
    ori = (point_orientation[:, :, 0].reshape(Nb, _T, P)
           .transpose(0, 2, 1).reshape(Nb, 1, P * _T))

    cidx = (polygon_type * 8 + polygon_tl_status * 2
            + polygon_on_route).reshape(N, 1).astype(jnp.int32)
    hassl = polygon_has_speed_limit.reshape(N, 1).astype(f32)
    slval = polygon_speed_limit.reshape(N, 1)

    d1 = pe_W1.shape[1]
    d2 = pe_W2.shape[1]
    din = pe_W1.shape[0]
    e1 = jnp.zeros((1, din + 1), f32).at[0, din].set(1.0)
    W1a = jnp.concatenate(
        [jnp.concatenate([pe_W1.T, pe_b1[:, None]], axis=1), e1],
        axis=0).astype(bf16)
    e2 = jnp.zeros((1, d2 + 1), f32).at[0, d2].set(1.0)
    W2af = jnp.concatenate(
        [jnp.concatenate([pe_W2.T, pe_b2[:, None]], axis=1),
         jnp.zeros((1, d1 + 1), f32).at[0, d1].set(1.0)],
        axis=0)
    W2a = W2af.astype(bf16)
    W3aa = jnp.concatenate(
        [jnp.concatenate([pe_W3[:d2].T, jnp.zeros((d2, 1), f32)], axis=1),
         jnp.zeros((1, d2 + 1), f32)], axis=0)
    W3aC = (W3aa @ W2af).astype(bf16)
    W3ba = jnp.concatenate(
        [jnp.concatenate([pe_W3[d2:].T, pe_b3[:, None]], axis=1),
         e2], axis=0).astype(bf16)
    W4a = jnp.concatenate([pe_W4.T, pe_b4[:, None]],
                          axis=1).astype(bf16)

    ctab = (type_emb[:, None, None, :] + tl_emb[None, :, None, :]
            + on_route_emb[None, None, :, :]).reshape(24, dim)

    slb1 = sl_b1.reshape(1, -1)
    slb2 = sl_b2.reshape(1, -1)
    slW2b = sl_W2

    grid = (Nb,)
    PT = P * _T

    def pmspec():
        return pl.BlockSpec((1, 1, PT), lambda i: (i, 0, 0))

    def tok():
        return pl.BlockSpec((_T, 1), lambda i: (i, 0))

    def full(a):
        return pl.BlockSpec(a.shape, lambda i: (0,) * a.ndim)

    in_specs = [
        pl.BlockSpec((1, 5, PT), lambda i: (i, 0, 0)), pmspec(),
        tok(), tok(), tok(),
        full(W1a), full(W2a), full(W3aC), full(W3ba), full(W4a),
        full(sl_W1), full(slb1), full(slW2b), full(slb2),
        full(ctab), full(unknown_speed_emb),
    ]

    out = pl.pallas_call(
        _me_kernel,
        grid=grid,
        in_specs=in_specs,
        out_specs=pl.BlockSpec((_T, dim), lambda i: (i, 0)),
        out_shape=jax.ShapeDtypeStruct((N, dim), jnp.float32),
    )(comps, ori, cidx, hassl, slval,
      W1a, W2a, W3aC, W3ba, W4a,
      sl_W1, slb1, slW2b, slb2, ctab, unknown_speed_emb)

    return out.reshape(bs, M, dim)
